# Initial kernel scaffold; baseline (speedup 1.0000x reference)
import jax
import jax.numpy as jnp
from jax.experimental import pallas as pl


def kernel(x, Wg, bg, W1, b1, W2, b2):
    raise NotImplementedError("write your pallas kernel here")


# speedup vs baseline: 1.2345x; 1.2345x over previous
import functools

import jax
import jax.numpy as jnp
from jax import lax
from jax.experimental import pallas as pl
from jax.experimental.pallas import tpu as pltpu
from jax.experimental.pallas import tpu_sc as plsc

T, D, FF, E, K = 2048, 1024, 2048, 8, 2
BLK = 256
N_PAD = T * K + E * BLK
NB = N_PAD // BLK

NC, NS = 2, 16
NW = NC * NS
TPW = T // NW
CHUNK = 32


def _gate_body(x_ref, wg_ref, bg_ref, idx_ref, vals_ref, dest_ref, be_ref,
               v0x_ref, v1x_ref):
    xf = x_ref[...]
    logits = lax.dot_general(xf, wg_ref[...], (((1,), (1,)), ((), ())),
                             preferred_element_type=jnp.float32)
    logits = logits + bg_ref[...]
    m = jnp.max(logits, axis=1, keepdims=True)
    p = jnp.exp(logits - m)
    scores = p / jnp.sum(p, axis=1, keepdims=True)

    iota_e = lax.broadcasted_iota(jnp.int32, (T, E), 1)
    m1 = jnp.max(scores, axis=1, keepdims=True)
    i1 = jnp.min(jnp.where(scores == m1, iota_e, E), axis=1, keepdims=True)
    sel1 = iota_e == i1
    masked = jnp.where(sel1, -1.0, scores)
    m2 = jnp.max(masked, axis=1, keepdims=True)
    i2 = jnp.min(jnp.where(masked == m2, iota_e, E), axis=1, keepdims=True)
    sel2 = iota_e == i2

    idx_ref[...] = jnp.concatenate([i1, i2], axis=1)
    vals_ref[...] = jnp.concatenate([m1, m2], axis=1)
    zeros16 = jnp.zeros((T, 16), jnp.float32)
    v0x_ref[...] = m1 + zeros16
    v1x_ref[...] = m2 + zeros16

    m0f = sel1.astype(jnp.float32)
    m1f = sel2.astype(jnp.float32)
    rowsum = m0f + m1f
    ti = lax.broadcasted_iota(jnp.int32, (T, T), 0)
    tj = lax.broadcasted_iota(jnp.int32, (T, T), 1)
    tri = (tj < ti).astype(jnp.float32)
    cum_excl = lax.dot_general(tri, rowsum, (((1,), (0,)), ((), ())),
                               preferred_element_type=jnp.float32)
    counts = jnp.sum(rowsum, axis=0, keepdims=True)
    cnt_pad = jnp.floor((counts + (BLK - 1)) * (1.0 / BLK)) * BLK
    ei = lax.broadcasted_iota(jnp.int32, (E, E), 0)
    ej = lax.broadcasted_iota(jnp.int32, (E, E), 1)
    tri_e = (ei < ej).astype(jnp.float32)
    pad_off = lax.dot_general(cnt_pad, tri_e, (((1,), (0,)), ((), ())),
                              preferred_element_type=jnp.float32)
    base = pad_off + cum_excl
    d0 = jnp.sum(jnp.where(sel1, base, 0.0), axis=1, keepdims=True)
    d1 = jnp.sum(jnp.where(sel2, base, 0.0), axis=1, keepdims=True)
    dest_ref[...] = jnp.concatenate([d0, d1], axis=1).astype(jnp.int32)

    pos = lax.broadcasted_iota(jnp.int32, (NB, 1), 0).astype(jnp.float32) * BLK
    owners = jnp.sum((pad_off <= pos).astype(jnp.int32), axis=1, keepdims=True) - 1
    be_ref[...] = owners


def _gate(xf, Wg, bg):
    return pl.pallas_call(
        _gate_body,
        out_shape=(
            jax.ShapeDtypeStruct((T, K), jnp.int32),
            jax.ShapeDtypeStruct((T, K), jnp.float32),
            jax.ShapeDtypeStruct((T, K), jnp.int32),
            jax.ShapeDtypeStruct((NB, 1), jnp.int32),
            jax.ShapeDtypeStruct((T, 16), jnp.float32),
            jax.ShapeDtypeStruct((T, 16), jnp.float32),
        ),
    )(xf, Wg, bg.reshape(1, E))


def _dispatch_body(x_hbm, d0_hbm, d1_hbm, xs_hbm, rows_v, i0_v, i1_v, sem):
    wid = lax.axis_index("s") * NC + lax.axis_index("c")
    t0 = wid * TPW
    pltpu.sync_copy(x_hbm.at[pl.ds(t0, TPW)], rows_v)
    pltpu.sync_copy(d0_hbm.at[pl.ds(t0, TPW)], i0_v)
    pltpu.sync_copy(d1_hbm.at[pl.ds(t0, TPW)], i1_v)
    pltpu.async_copy(rows_v, xs_hbm.at[i0_v], sem).wait()
    pltpu.async_copy(rows_v, xs_hbm.at[i1_v], sem).wait()


@functools.cache
def _make_dispatch():
    return pl.kernel(
        _dispatch_body,
        out_type=jax.ShapeDtypeStruct((N_PAD, D), jnp.float32),
        mesh=plsc.VectorSubcoreMesh(core_axis_name="c", subcore_axis_name="s",
                                    num_cores=NC, num_subcores=NS),
        scratch_types=[
            pltpu.VMEM((TPW, D), jnp.float32),
            pltpu.VMEM((TPW,), jnp.int32),
            pltpu.VMEM((TPW,), jnp.int32),
            pltpu.SemaphoreType.DMA,
        ],
    )


def _ffn_body(be_ref, xs_ref, w1_ref, b1_ref, w2_ref, b2_ref, ys_ref):
    del be_ref
    xb = xs_ref[...]
    h = lax.dot_general(xb, w1_ref[0], (((1,), (1,)), ((), ())),
                        preferred_element_type=jnp.float32)
    h = h + b1_ref[0]
    h = jnp.where(h >= 0, h, 0.1 * h)
    y = lax.dot_general(h, w2_ref[0], (((1,), (1,)), ((), ())),
                        preferred_element_type=jnp.float32)
    ys_ref[...] = y + b2_ref[0]


def _ffn(be, xs, W1, b1, W2, b2):
    grid_spec = pltpu.PrefetchScalarGridSpec(
        num_scalar_prefetch=1,
        grid=(NB,),
        in_specs=[
            pl.BlockSpec((BLK, D), lambda b, be: (b, 0)),
            pl.BlockSpec((1, FF, D), lambda b, be: (be[b], 0, 0)),
            pl.BlockSpec((1, 1, FF), lambda b, be: (be[b], 0, 0)),
            pl.BlockSpec((1, D, FF), lambda b, be: (be[b], 0, 0)),
            pl.BlockSpec((1, 1, D), lambda b, be: (be[b], 0, 0)),
        ],
        out_specs=pl.BlockSpec((BLK, D), lambda b, be: (b, 0)),
    )
    return pl.pallas_call(
        _ffn_body,
        grid_spec=grid_spec,
        out_shape=jax.ShapeDtypeStruct((N_PAD, D), jnp.float32),
    )(be, xs, W1, b1.reshape(E, 1, FF), W2, b2.reshape(E, 1, D))


def _combine_body(ys_hbm, d0_hbm, d1_hbm, v0x_hbm, v1x_hbm, out_hbm,
                  g0_v, g1_v, i0_v, i1_v, v0x_v, v1x_v, sem):
    wid = lax.axis_index("s") * NC + lax.axis_index("c")
    for c in range(TPW // CHUNK):
        t0 = wid * TPW + c * CHUNK
        pltpu.sync_copy(d0_hbm.at[pl.ds(t0, CHUNK)], i0_v)
        pltpu.sync_copy(d1_hbm.at[pl.ds(t0, CHUNK)], i1_v)
        pltpu.sync_copy(v0x_hbm.at[pl.ds(t0, CHUNK)], v0x_v)
        pltpu.sync_copy(v1x_hbm.at[pl.ds(t0, CHUNK)], v1x_v)
        pltpu.async_copy(ys_hbm.at[i0_v], g0_v, sem).wait()
        pltpu.async_copy(ys_hbm.at[i1_v], g1_v, sem).wait()

        def row_body(r, carry):
            vs0 = v0x_v[r, :]
            vs1 = v1x_v[r, :]

            def col_body(cc, carry2):
                sl = pl.ds(cc * 16, 16)
                g0_v[r, sl] = g0_v[r, sl] * vs0 + g1_v[r, sl] * vs1
                return carry2

            return lax.fori_loop(0, D // 16, col_body, carry)

        lax.fori_loop(0, CHUNK, row_body, 0)
        pltpu.sync_copy(g0_v, out_hbm.at[pl.ds(t0, CHUNK)])


@functools.cache
def _make_combine():
    return pl.kernel(
        _combine_body,
        out_type=jax.ShapeDtypeStruct((T, D), jnp.float32),
        mesh=plsc.VectorSubcoreMesh(core_axis_name="c", subcore_axis_name="s",
                                    num_cores=NC, num_subcores=NS),
        scratch_types=[
            pltpu.VMEM((CHUNK, D), jnp.float32),
            pltpu.VMEM((CHUNK, D), jnp.float32),
            pltpu.VMEM((CHUNK,), jnp.int32),
            pltpu.VMEM((CHUNK,), jnp.int32),
            pltpu.VMEM((CHUNK, 16), jnp.float32),
            pltpu.VMEM((CHUNK, 16), jnp.float32),
            pltpu.SemaphoreType.DMA,
        ],
    )


def kernel(x, Wg, bg, W1, b1, W2, b2):
    b, s, d = x.shape
    xf = x.reshape(T, D)
    topk_idx, topk_vals, dest, be, v0x, v1x = _gate(xf, Wg, bg)
    d0, d1 = dest[:, 0], dest[:, 1]
    xs = _make_dispatch()(xf, d0, d1)
    ys = _ffn(be[:, 0], xs, W1, b1, W2, b2)
    out = _make_combine()(ys, d0, d1, v0x, v1x)
    return out.reshape(b, s, d), topk_idx, topk_vals


# speedup vs baseline: 1.3669x; 1.1073x over previous
import functools

import jax
import jax.numpy as jnp
from jax import lax
from jax.experimental import pallas as pl
from jax.experimental.pallas import tpu as pltpu
from jax.experimental.pallas import tpu_sc as plsc

T, D, FF, E, K = 2048, 1024, 2048, 8, 2
BLK = 256
N_PAD = T * K + E * BLK
NB = N_PAD // BLK

NC, NS = 2, 16
NW = NC * NS
TPW = T // NW
CHUNK = 32


def _gate_body(x_ref, wg_ref, bg_ref, idx_ref, vals_ref, dest_ref, be_ref,
               v0x_ref, v1x_ref):
    xf = x_ref[...]
    logits = lax.dot_general(xf, wg_ref[...], (((1,), (1,)), ((), ())),
                             preferred_element_type=jnp.float32)
    logits = logits + bg_ref[...]
    m = jnp.max(logits, axis=1, keepdims=True)
    p = jnp.exp(logits - m)
    scores = p / jnp.sum(p, axis=1, keepdims=True)

    iota_e = lax.broadcasted_iota(jnp.int32, (T, E), 1)
    m1 = jnp.max(scores, axis=1, keepdims=True)
    i1 = jnp.min(jnp.where(scores == m1, iota_e, E), axis=1, keepdims=True)
    sel1 = iota_e == i1
    masked = jnp.where(sel1, -1.0, scores)
    m2 = jnp.max(masked, axis=1, keepdims=True)
    i2 = jnp.min(jnp.where(masked == m2, iota_e, E), axis=1, keepdims=True)
    sel2 = iota_e == i2

    idx_ref[...] = jnp.concatenate([i1, i2], axis=1)
    vals_ref[...] = jnp.concatenate([m1, m2], axis=1)
    zeros16 = jnp.zeros((T, 128), jnp.float32)
    v0x_ref[...] = m1 + zeros16
    v1x_ref[...] = m2 + zeros16

    m0f = sel1.astype(jnp.float32)
    m1f = sel2.astype(jnp.float32)
    rowsum = m0f + m1f
    ti = lax.broadcasted_iota(jnp.int32, (T, T), 0)
    tj = lax.broadcasted_iota(jnp.int32, (T, T), 1)
    tri = (tj < ti).astype(jnp.float32)
    cum_excl = lax.dot_general(tri, rowsum, (((1,), (0,)), ((), ())),
                               preferred_element_type=jnp.float32)
    counts = jnp.sum(rowsum, axis=0, keepdims=True)
    cnt_pad = jnp.floor((counts + (BLK - 1)) * (1.0 / BLK)) * BLK
    ei = lax.broadcasted_iota(jnp.int32, (E, E), 0)
    ej = lax.broadcasted_iota(jnp.int32, (E, E), 1)
    tri_e = (ei < ej).astype(jnp.float32)
    pad_off = lax.dot_general(cnt_pad, tri_e, (((1,), (0,)), ((), ())),
                              preferred_element_type=jnp.float32)
    base = pad_off + cum_excl
    d0 = jnp.sum(jnp.where(sel1, base, 0.0), axis=1, keepdims=True)
    d1 = jnp.sum(jnp.where(sel2, base, 0.0), axis=1, keepdims=True)
    dest_ref[...] = jnp.concatenate([d0, d1], axis=1).astype(jnp.int32)

    pos = lax.broadcasted_iota(jnp.int32, (NB, 1), 0).astype(jnp.float32) * BLK
    owners = jnp.sum((pad_off <= pos).astype(jnp.int32), axis=1, keepdims=True) - 1
    be_ref[...] = owners


def _gate(xf, Wg, bg):
    return pl.pallas_call(
        _gate_body,
        out_shape=(
            jax.ShapeDtypeStruct((T, K), jnp.int32),
            jax.ShapeDtypeStruct((T, K), jnp.float32),
            jax.ShapeDtypeStruct((T, K), jnp.int32),
            jax.ShapeDtypeStruct((NB, 1), jnp.int32),
            jax.ShapeDtypeStruct((T, 128), jnp.float32),
            jax.ShapeDtypeStruct((T, 128), jnp.float32),
        ),
    )(xf, Wg, bg.reshape(1, E))


def _dispatch_body(x_hbm, d0_hbm, d1_hbm, v0x_hbm, v1x_hbm, xs_hbm, wx_hbm,
                   rows_v, i0_v, i1_v, w0_v, w1_v, sem):
    wid = lax.axis_index("s") * NC + lax.axis_index("c")
    t0 = wid * TPW
    pltpu.sync_copy(x_hbm.at[pl.ds(t0, TPW)], rows_v)
    pltpu.sync_copy(d0_hbm.at[pl.ds(t0, TPW)], i0_v)
    pltpu.sync_copy(d1_hbm.at[pl.ds(t0, TPW)], i1_v)
    pltpu.sync_copy(v0x_hbm.at[pl.ds(t0, TPW)], w0_v)
    pltpu.sync_copy(v1x_hbm.at[pl.ds(t0, TPW)], w1_v)
    c0 = pltpu.async_copy(rows_v, xs_hbm.at[i0_v], sem)
    c1 = pltpu.async_copy(rows_v, xs_hbm.at[i1_v], sem)
    c2 = pltpu.async_copy(w0_v, wx_hbm.at[i0_v], sem)
    c3 = pltpu.async_copy(w1_v, wx_hbm.at[i1_v], sem)
    c0.wait()
    c1.wait()
    c2.wait()
    c3.wait()


@functools.cache
def _make_dispatch():
    return pl.kernel(
        _dispatch_body,
        out_type=(
            jax.ShapeDtypeStruct((N_PAD, D), jnp.float32),
            jax.ShapeDtypeStruct((N_PAD, 128), jnp.float32),
        ),
        mesh=plsc.VectorSubcoreMesh(core_axis_name="c", subcore_axis_name="s",
                                    num_cores=NC, num_subcores=NS),
        scratch_types=[
            pltpu.VMEM((TPW, D), jnp.float32),
            pltpu.VMEM((TPW,), jnp.int32),
            pltpu.VMEM((TPW,), jnp.int32),
            pltpu.VMEM((TPW, 128), jnp.float32),
            pltpu.VMEM((TPW, 128), jnp.float32),
            pltpu.SemaphoreType.DMA,
        ],
    )


def _ffn_body(be_ref, xs_ref, w1_ref, b1_ref, w2_ref, b2_ref, wx_ref, ys_ref):
    del be_ref
    xb = xs_ref[...]
    h = lax.dot_general(xb, w1_ref[0], (((1,), (1,)), ((), ())),
                        preferred_element_type=jnp.float32)
    h = h + b1_ref[0]
    h = jnp.where(h >= 0, h, 0.1 * h)
    y = lax.dot_general(h, w2_ref[0], (((1,), (1,)), ((), ())),
                        preferred_element_type=jnp.float32)
    ys_ref[...] = (y + b2_ref[0]) * wx_ref[:, 0:1]


def _ffn(be, xs, W1, b1, W2, b2, wx):
    grid_spec = pltpu.PrefetchScalarGridSpec(
        num_scalar_prefetch=1,
        grid=(NB,),
        in_specs=[
            pl.BlockSpec((BLK, D), lambda b, be: (b, 0)),
            pl.BlockSpec((1, FF, D), lambda b, be: (be[b], 0, 0)),
            pl.BlockSpec((1, 1, FF), lambda b, be: (be[b], 0, 0)),
            pl.BlockSpec((1, D, FF), lambda b, be: (be[b], 0, 0)),
            pl.BlockSpec((1, 1, D), lambda b, be: (be[b], 0, 0)),
            pl.BlockSpec((BLK, 128), lambda b, be: (b, 0)),
        ],
        out_specs=pl.BlockSpec((BLK, D), lambda b, be: (b, 0)),
    )
    return pl.pallas_call(
        _ffn_body,
        grid_spec=grid_spec,
        out_shape=jax.ShapeDtypeStruct((N_PAD, D), jnp.float32),
    )(be, xs, W1, b1.reshape(E, 1, FF), W2, b2.reshape(E, 1, D), wx)


def _combine_body(ys_hbm, d0_hbm, d1_hbm, out_hbm,
                  g0_v, g1_v, i0_v, i1_v, sem):
    wid = lax.axis_index("s") * NC + lax.axis_index("c")
    for c in range(TPW // CHUNK):
        t0 = wid * TPW + c * CHUNK
        pltpu.sync_copy(d0_hbm.at[pl.ds(t0, CHUNK)], i0_v)
        pltpu.sync_copy(d1_hbm.at[pl.ds(t0, CHUNK)], i1_v)
        c0 = pltpu.async_copy(ys_hbm.at[i0_v], g0_v, sem)
        c1 = pltpu.async_copy(ys_hbm.at[i1_v], g1_v, sem)
        c0.wait()
        c1.wait()

        def row_body(r, carry):
            for cc in range(D // 16):
                sl = pl.ds(cc * 16, 16)
                g0_v[r, sl] = g0_v[r, sl] + g1_v[r, sl]
            return carry

        lax.fori_loop(0, CHUNK, row_body, 0)
        pltpu.sync_copy(g0_v, out_hbm.at[pl.ds(t0, CHUNK)])


@functools.cache
def _make_combine():
    return pl.kernel(
        _combine_body,
        out_type=jax.ShapeDtypeStruct((T, D), jnp.float32),
        mesh=plsc.VectorSubcoreMesh(core_axis_name="c", subcore_axis_name="s",
                                    num_cores=NC, num_subcores=NS),
        scratch_types=[
            pltpu.VMEM((CHUNK, D), jnp.float32),
            pltpu.VMEM((CHUNK, D), jnp.float32),
            pltpu.VMEM((CHUNK,), jnp.int32),
            pltpu.VMEM((CHUNK,), jnp.int32),
            pltpu.SemaphoreType.DMA,
        ],
    )


def kernel(x, Wg, bg, W1, b1, W2, b2):
    b, s, d = x.shape
    xf = x.reshape(T, D)
    topk_idx, topk_vals, dest, be, v0x, v1x = _gate(xf, Wg, bg)
    d0, d1 = dest[:, 0], dest[:, 1]
    xs, wx = _make_dispatch()(xf, d0, d1, v0x, v1x)
    ys = _ffn(be[:, 0], xs, W1, b1, W2, b2, wx)
    out = _make_combine()(ys, d0, d1)
    return out.reshape(b, s, d), topk_idx, topk_vals


# speedup vs baseline: 1.4342x; 1.0492x over previous
import functools

import jax
import jax.numpy as jnp
from jax import lax
from jax.experimental import pallas as pl
from jax.experimental.pallas import tpu as pltpu
from jax.experimental.pallas import tpu_sc as plsc

T, D, FF, E, K = 2048, 1024, 2048, 8, 2
BLK = 256
N_PAD = T * K + E * BLK
NB = N_PAD // BLK

NC, NS = 2, 16
NW = NC * NS
TPW = T // NW
CHUNK = 32


def _gate_body(x_ref, wg_ref, bg_ref, idx_ref, vals_ref, dest_ref, be_ref,
               v0x_ref, v1x_ref):
    xf = x_ref[...]
    logits = lax.dot_general(xf, wg_ref[...], (((1,), (1,)), ((), ())),
                             preferred_element_type=jnp.float32)
    logits = logits + bg_ref[...]
    m = jnp.max(logits, axis=1, keepdims=True)
    p = jnp.exp(logits - m)
    scores = p / jnp.sum(p, axis=1, keepdims=True)

    iota_e = lax.broadcasted_iota(jnp.int32, (T, E), 1)
    m1 = jnp.max(scores, axis=1, keepdims=True)
    i1 = jnp.min(jnp.where(scores == m1, iota_e, E), axis=1, keepdims=True)
    sel1 = iota_e == i1
    masked = jnp.where(sel1, -1.0, scores)
    m2 = jnp.max(masked, axis=1, keepdims=True)
    i2 = jnp.min(jnp.where(masked == m2, iota_e, E), axis=1, keepdims=True)
    sel2 = iota_e == i2

    idx_ref[...] = jnp.concatenate([i1, i2], axis=1)
    vals_ref[...] = jnp.concatenate([m1, m2], axis=1)
    zeros16 = jnp.zeros((T, 128), jnp.float32)
    v0x_ref[...] = m1 + zeros16
    v1x_ref[...] = m2 + zeros16

    m0f = sel1.astype(jnp.float32)
    m1f = sel2.astype(jnp.float32)
    rowsum = m0f + m1f
    ti = lax.broadcasted_iota(jnp.int32, (T, T), 0)
    tj = lax.broadcasted_iota(jnp.int32, (T, T), 1)
    tri = (tj < ti).astype(jnp.float32)
    cum_excl = lax.dot_general(tri, rowsum, (((1,), (0,)), ((), ())),
                               preferred_element_type=jnp.float32)
    counts = jnp.sum(rowsum, axis=0, keepdims=True)
    cnt_pad = jnp.floor((counts + (BLK - 1)) * (1.0 / BLK)) * BLK
    ei = lax.broadcasted_iota(jnp.int32, (E, E), 0)
    ej = lax.broadcasted_iota(jnp.int32, (E, E), 1)
    tri_e = (ei < ej).astype(jnp.float32)
    pad_off = lax.dot_general(cnt_pad, tri_e, (((1,), (0,)), ((), ())),
                              preferred_element_type=jnp.float32)
    base = pad_off + cum_excl
    d0 = jnp.sum(jnp.where(sel1, base, 0.0), axis=1, keepdims=True)
    d1 = jnp.sum(jnp.where(sel2, base, 0.0), axis=1, keepdims=True)
    dest_ref[...] = jnp.concatenate([d0, d1], axis=1).astype(jnp.int32)

    pos = lax.broadcasted_iota(jnp.int32, (NB, 1), 0).astype(jnp.float32) * BLK
    owners = jnp.sum((pad_off <= pos).astype(jnp.int32), axis=1, keepdims=True) - 1
    nb_real = jnp.sum(cnt_pad, axis=1, keepdims=True) * (1.0 / BLK)
    be_ref[...] = jnp.concatenate(
        [owners, jnp.broadcast_to(nb_real.astype(jnp.int32), (8, 1))], axis=0)


def _gate(xf, Wg, bg):
    return pl.pallas_call(
        _gate_body,
        out_shape=(
            jax.ShapeDtypeStruct((T, K), jnp.int32),
            jax.ShapeDtypeStruct((T, K), jnp.float32),
            jax.ShapeDtypeStruct((T, K), jnp.int32),
            jax.ShapeDtypeStruct((NB + 8, 1), jnp.int32),
            jax.ShapeDtypeStruct((T, 128), jnp.float32),
            jax.ShapeDtypeStruct((T, 128), jnp.float32),
        ),
    )(xf, Wg, bg.reshape(1, E))


def _dispatch_body(x_hbm, d0_hbm, d1_hbm, v0x_hbm, v1x_hbm, xs_hbm, wx_hbm,
                   rows_v, i0_v, i1_v, w0_v, w1_v, sem):
    wid = lax.axis_index("s") * NC + lax.axis_index("c")
    t0 = wid * TPW
    pltpu.sync_copy(x_hbm.at[pl.ds(t0, TPW)], rows_v)
    pltpu.sync_copy(d0_hbm.at[pl.ds(t0, TPW)], i0_v)
    pltpu.sync_copy(d1_hbm.at[pl.ds(t0, TPW)], i1_v)
    pltpu.sync_copy(v0x_hbm.at[pl.ds(t0, TPW)], w0_v)
    pltpu.sync_copy(v1x_hbm.at[pl.ds(t0, TPW)], w1_v)
    c0 = pltpu.async_copy(rows_v, xs_hbm.at[i0_v], sem)
    c1 = pltpu.async_copy(rows_v, xs_hbm.at[i1_v], sem)
    c2 = pltpu.async_copy(w0_v, wx_hbm.at[i0_v], sem)
    c3 = pltpu.async_copy(w1_v, wx_hbm.at[i1_v], sem)
    c0.wait()
    c1.wait()
    c2.wait()
    c3.wait()


@functools.cache
def _make_dispatch():
    return pl.kernel(
        _dispatch_body,
        out_type=(
            jax.ShapeDtypeStruct((N_PAD, D), jnp.float32),
            jax.ShapeDtypeStruct((N_PAD, 128), jnp.float32),
        ),
        mesh=plsc.VectorSubcoreMesh(core_axis_name="c", subcore_axis_name="s",
                                    num_cores=NC, num_subcores=NS),
        scratch_types=[
            pltpu.VMEM((TPW, D), jnp.float32),
            pltpu.VMEM((TPW,), jnp.int32),
            pltpu.VMEM((TPW,), jnp.int32),
            pltpu.VMEM((TPW, 128), jnp.float32),
            pltpu.VMEM((TPW, 128), jnp.float32),
            pltpu.SemaphoreType.DMA,
        ],
    )


def _ffn_body(be_ref, xs_ref, w1_ref, b1_ref, w2_ref, b2_ref, wx_ref, ys_ref):
    @pl.when(pl.program_id(0) < be_ref[NB])
    def _():
        xb = xs_ref[...].astype(jnp.bfloat16)
        h = lax.dot_general(xb, w1_ref[0].astype(jnp.bfloat16),
                            (((1,), (1,)), ((), ())),
                            preferred_element_type=jnp.float32)
        h = h + b1_ref[0]
        h = jnp.where(h >= 0, h, 0.1 * h)
        y = lax.dot_general(h.astype(jnp.bfloat16),
                            w2_ref[0].astype(jnp.bfloat16),
                            (((1,), (1,)), ((), ())),
                            preferred_element_type=jnp.float32)
        ys_ref[...] = (y + b2_ref[0]) * wx_ref[:, 0:1]


def _ffn(be, xs, W1, b1, W2, b2, wx):
    grid_spec = pltpu.PrefetchScalarGridSpec(
        num_scalar_prefetch=1,
        grid=(NB,),
        in_specs=[
            pl.BlockSpec((BLK, D), lambda b, be: (b, 0)),
            pl.BlockSpec((1, FF, D), lambda b, be: (be[b], 0, 0)),
            pl.BlockSpec((1, 1, FF), lambda b, be: (be[b], 0, 0)),
            pl.BlockSpec((1, D, FF), lambda b, be: (be[b], 0, 0)),
            pl.BlockSpec((1, 1, D), lambda b, be: (be[b], 0, 0)),
            pl.BlockSpec((BLK, 128), lambda b, be: (b, 0)),
        ],
        out_specs=pl.BlockSpec((BLK, D), lambda b, be: (b, 0)),
    )
    return pl.pallas_call(
        _ffn_body,
        grid_spec=grid_spec,
        out_shape=jax.ShapeDtypeStruct((N_PAD, D), jnp.float32),
    )(be, xs, W1, b1.reshape(E, 1, FF), W2, b2.reshape(E, 1, D), wx)


def _combine_body(ys_hbm, d0_hbm, d1_hbm, out_hbm,
                  g0_v, g1_v, i0_v, i1_v, sem):
    wid = lax.axis_index("s") * NC + lax.axis_index("c")
    for c in range(TPW // CHUNK):
        t0 = wid * TPW + c * CHUNK
        pltpu.sync_copy(d0_hbm.at[pl.ds(t0, CHUNK)], i0_v)
        pltpu.sync_copy(d1_hbm.at[pl.ds(t0, CHUNK)], i1_v)
        c0 = pltpu.async_copy(ys_hbm.at[i0_v], g0_v, sem)
        c1 = pltpu.async_copy(ys_hbm.at[i1_v], g1_v, sem)
        c0.wait()
        c1.wait()

        def row_body(r, carry):
            for cc in range(D // 16):
                sl = pl.ds(cc * 16, 16)
                g0_v[r, sl] = g0_v[r, sl] + g1_v[r, sl]
            return carry

        lax.fori_loop(0, CHUNK, row_body, 0)
        pltpu.sync_copy(g0_v, out_hbm.at[pl.ds(t0, CHUNK)])


@functools.cache
def _make_combine():
    return pl.kernel(
        _combine_body,
        out_type=jax.ShapeDtypeStruct((T, D), jnp.float32),
        mesh=plsc.VectorSubcoreMesh(core_axis_name="c", subcore_axis_name="s",
                                    num_cores=NC, num_subcores=NS),
        scratch_types=[
            pltpu.VMEM((CHUNK, D), jnp.float32),
            pltpu.VMEM((CHUNK, D), jnp.float32),
            pltpu.VMEM((CHUNK,), jnp.int32),
            pltpu.VMEM((CHUNK,), jnp.int32),
            pltpu.SemaphoreType.DMA,
        ],
    )


def kernel(x, Wg, bg, W1, b1, W2, b2):
    b, s, d = x.shape
    xf = x.reshape(T, D)
    topk_idx, topk_vals, dest, be, v0x, v1x = _gate(xf, Wg, bg)
    d0, d1 = dest[:, 0], dest[:, 1]
    xs, wx = _make_dispatch()(xf, d0, d1, v0x, v1x)
    ys = _ffn(be[:, 0], xs, W1, b1, W2, b2, wx)
    out = _make_combine()(ys, d0, d1)
    return out.reshape(b, s, d), topk_idx, topk_vals
